# Initial kernel scaffold; baseline (speedup 1.0000x reference)
import jax
import jax.numpy as jnp
from jax.experimental import pallas as pl


def kernel(cell_phi, cell_grad, cells_node, cells_index, centroid, mesh_pos):
    raise NotImplementedError("write your pallas kernel here")


# speedup vs baseline: 68.0940x; 68.0940x over previous
import dataclasses
import functools

import jax
import jax.numpy as jnp
from jax import lax
from jax.experimental import pallas as pl
from jax.experimental.pallas import tpu as pltpu
from jax.experimental.pallas import tpu_sc as plsc

NC = 2
NS = 16
NW = NC * NS
L = 16

N_NODES = 100000
N_CELLS = 200000
E = 600000
C = 3

B = 1920
BLOCKS = 10
EPW = B * BLOCKS
E_PAD = EPW * NW
N_ACC = 100096
ACC_PW = N_ACC // NS
ZROWS = ACC_PW // 4
ROWS_PW = N_ACC // NW


def _rsqrt_nr(x):
    xh = x * 0.5
    i = plsc.bitcast(x, jnp.int32)
    i = jnp.int32(0x5F3759DF) - (i >> 1)
    y = plsc.bitcast(i, jnp.float32)
    y = y * (1.5 - xh * y * y)
    y = y * (1.5 - xh * y * y)
    y = y * (1.5 - xh * y * y)
    return y


def _scatter_body(ntab_hbm, ctab_hbm, idxn_hbm, idxc_hbm, part_hbm,
                  idxn_v, idxc_v, nrows_v, crows_v, out_v, zbuf_v,
                  acc_sh, sem0, sem1):
    core = lax.axis_index("c")
    sid = lax.axis_index("s")
    wid = core * NS + sid
    iota = lax.iota(jnp.int32, L)

    zero = jnp.zeros((L,), jnp.float32)
    zr = iota >> 2
    zc = iota & 3

    @pl.loop(0, ZROWS * 4, step=L)
    def _(i):
        plsc.store_scatter(zbuf_v, [zr + (i >> 2), zc], zero)

    for z in range(4):
        pltpu.sync_copy(zbuf_v, acc_sh.at[pl.ds(sid * ACC_PW + z * ZROWS, ZROWS)])

    plsc.subcore_barrier()

    col = [jnp.full((L,), j, jnp.int32) for j in range(8)]

    for b in range(BLOCKS):
        base = wid * EPW + b * B
        cp0 = pltpu.async_copy(idxn_hbm.at[pl.ds(base, B)], idxn_v, sem0)
        cp1 = pltpu.async_copy(idxc_hbm.at[pl.ds(base, B)], idxc_v, sem1)
        cp0.wait()
        cp1.wait()
        cp2 = pltpu.async_copy(ntab_hbm.at[idxn_v], nrows_v, sem0)
        cp3 = pltpu.async_copy(ctab_hbm.at[idxc_v], crows_v, sem1)
        cp2.wait()
        cp3.wait()

        @pl.loop(0, B, step=L)
        def _(o):
            rows = o + iota
            mpx = plsc.load_gather(nrows_v, [rows, col[0]])
            mpy = plsc.load_gather(nrows_v, [rows, col[1]])
            cx = plsc.load_gather(crows_v, [rows, col[0]])
            cy = plsc.load_gather(crows_v, [rows, col[1]])
            dx = mpx - cx
            dy = mpy - cy
            w = _rsqrt_nr(dx * dx + dy * dy)
            plsc.store_scatter(out_v, [rows, col[3]], w)
            for ch in range(C):
                gx = plsc.load_gather(nrows_v, [rows, col[2 + 2 * ch]])
                gy = plsc.load_gather(nrows_v, [rows, col[3 + 2 * ch]])
                phi = plsc.load_gather(crows_v, [rows, col[2 + ch]])
                agg = (phi + dx * gx + dy * gy) * w
                plsc.store_scatter(out_v, [rows, col[ch]], agg)

        pltpu.sync_copy(out_v, acc_sh.at[idxn_v], add=True)

    plsc.subcore_barrier()

    r0 = sid * ACC_PW
    pltpu.sync_copy(acc_sh.at[pl.ds(r0, ACC_PW)],
                    part_hbm.at[core, pl.ds(r0, ACC_PW)])


def _combine_body(part_hbm, out_hbm, a0_v, a1_v, out_v):
    core = lax.axis_index("c")
    sid = lax.axis_index("s")
    wid = core * NS + sid
    r0 = wid * ROWS_PW
    pltpu.sync_copy(part_hbm.at[0, pl.ds(r0, ROWS_PW)], a0_v)
    pltpu.sync_copy(part_hbm.at[1, pl.ds(r0, ROWS_PW)], a1_v)

    iota = lax.iota(jnp.int32, L)
    col = [jnp.full((L,), j, jnp.int32) for j in range(4)]
    last = ROWS_PW - L

    @pl.loop(0, ((ROWS_PW + L - 1) // L) * L, step=L)
    def _(o):
        rows = jnp.minimum(o, last) + iota
        den = (plsc.load_gather(a0_v, [rows, col[3]])
               + plsc.load_gather(a1_v, [rows, col[3]]))
        rden = 1.0 / den
        for ch in range(C):
            num = (plsc.load_gather(a0_v, [rows, col[ch]])
                   + plsc.load_gather(a1_v, [rows, col[ch]]))
            plsc.store_scatter(out_v, [rows, col[ch]], num * rden)

    tail = N_NODES - (NW - 1) * ROWS_PW

    @pl.when(wid < NW - 1)
    def _():
        pltpu.sync_copy(out_v, out_hbm.at[pl.ds(r0, ROWS_PW)])

    @pl.when(wid == NW - 1)
    def _():
        pltpu.sync_copy(out_v.at[pl.ds(0, tail)],
                        out_hbm.at[pl.ds((NW - 1) * ROWS_PW, tail)])


_mesh = plsc.VectorSubcoreMesh(core_axis_name="c", subcore_axis_name="s")

_cp = pltpu.CompilerParams()
for _f, _v in (("needs_layout_passes", False), ("use_tc_tiling_on_sc", False)):
    if _f in pltpu.CompilerParams.__dataclass_fields__:
        _cp = dataclasses.replace(_cp, **{_f: _v})

_scatter_phase = functools.partial(
    pl.kernel,
    out_type=jax.ShapeDtypeStruct((NC, N_ACC, 4), jnp.float32),
    mesh=_mesh,
    compiler_params=_cp,
    scratch_types=[
        pltpu.VMEM((B,), jnp.int32),
        pltpu.VMEM((B,), jnp.int32),
        pltpu.VMEM((B, 8), jnp.float32),
        pltpu.VMEM((B, 8), jnp.float32),
        pltpu.VMEM((B, 4), jnp.float32),
        pltpu.VMEM((ZROWS, 4), jnp.float32),
        pltpu.VMEM_SHARED((N_ACC, 4), jnp.float32),
        pltpu.SemaphoreType.DMA,
        pltpu.SemaphoreType.DMA,
    ],
)

_combine_phase = functools.partial(
    pl.kernel,
    out_type=jax.ShapeDtypeStruct((N_NODES, C), jnp.float32),
    mesh=_mesh,
    compiler_params=_cp,
    scratch_types=[
        pltpu.VMEM((ROWS_PW, 4), jnp.float32),
        pltpu.VMEM((ROWS_PW, 4), jnp.float32),
        pltpu.VMEM((ROWS_PW, C), jnp.float32),
    ],
)


def kernel(cell_phi, cell_grad, cells_node, cells_index, centroid, mesh_pos):
    n = mesh_pos.shape[0]
    ntab = jnp.concatenate(
        [mesh_pos, cell_grad[:n].reshape(n, 2 * C)], axis=1)
    ntab = jnp.concatenate(
        [ntab, jnp.zeros((N_ACC - n, 8), jnp.float32)], axis=0)
    ctab = jnp.concatenate(
        [centroid, cell_phi, jnp.zeros((N_CELLS, 3), jnp.float32)], axis=1)

    pad = E_PAD - E
    idxn = jnp.concatenate(
        [cells_node, jnp.full((pad,), N_NODES, jnp.int32)])
    idxc = jnp.concatenate([cells_index, jnp.zeros((pad,), jnp.int32)])

    acc = _scatter_phase(_scatter_body)(ntab, ctab, idxn, idxc)
    return _combine_phase(_combine_body)(acc)


# speedup vs baseline: 110.0657x; 1.6164x over previous
import dataclasses
import functools

import jax
import jax.numpy as jnp
from jax import lax
from jax.experimental import pallas as pl
from jax.experimental.pallas import tpu as pltpu
from jax.experimental.pallas import tpu_sc as plsc

NC = 2
NS = 16
NW = NC * NS
L = 16

N_NODES = 100000
N_CELLS = 200000
E = 600000
C = 3
TRASH = N_NODES

B = 1920
BLOCKS = 10
EPW = B * BLOCKS
E_PAD = EPW * NW

NM = 100352
MPW = NM // NW
N_ACC = NM
ACC_PW = N_ACC // NS
ZCH = ACC_PW // 8

NCT = 200192
CPW = NCT // NW
CCH = CPW // 2
MCH = MPW // 2
ICH = EPW // 4

_mesh = plsc.VectorSubcoreMesh(core_axis_name="c", subcore_axis_name="s")
_cp = pltpu.CompilerParams()
for _f, _v in (("needs_layout_passes", False), ("use_tc_tiling_on_sc", False)):
    if _f in pltpu.CompilerParams.__dataclass_fields__:
        _cp = dataclasses.replace(_cp, **{_f: _v})

_f32 = jnp.float32
_i32 = jnp.int32


def _rsqrt_nr(x):
    xh = x * 0.5
    i = plsc.bitcast(x, _i32)
    i = jnp.int32(0x5F3759DF) - (i >> 1)
    y = plsc.bitcast(i, _f32)
    y = y * (1.5 - xh * y * y)
    y = y * (1.5 - xh * y * y)
    y = y * (1.5 - xh * y * y)
    return y


def _pack_body(mpx_hbm, mpy_hbm, cx_hbm, cy_hbm, p0_hbm, p1_hbm, p2_hbm,
               idxn_hbm, idxc_hbm,
               mtab_hbm, ctab_hbm, idxnp_hbm, idxcp_hbm,
               c0_v, c1_v, c2_v, c3_v, c4_v, ct_v, mx_v, my_v, mt_v, ix_v):
    core = lax.axis_index("c")
    sid = lax.axis_index("s")
    wid = core * NS + sid
    iota = lax.iota(_i32, L)
    cols = [jnp.full((L,), j, _i32) for j in range(8)]

    for s in range(2):
        cbase = wid * CPW + s * CCH
        for src, dst in ((cx_hbm, c0_v), (cy_hbm, c1_v), (p0_hbm, c2_v),
                         (p1_hbm, c3_v), (p2_hbm, c4_v)):
            pltpu.sync_copy(src.at[pl.ds(cbase, CCH)], dst)

        @pl.loop(0, CCH, step=L)
        def _(o):
            rows = o + iota
            for j, src_v in enumerate((c0_v, c1_v, c2_v, c3_v, c4_v)):
                plsc.store_scatter(ct_v, [rows, cols[j]], src_v[pl.ds(o, L)])

        pltpu.sync_copy(ct_v, ctab_hbm.at[pl.ds(cbase, CCH)])

    for s in range(2):
        nbase = wid * MPW + s * MCH
        pltpu.sync_copy(mpx_hbm.at[pl.ds(nbase, MCH)], mx_v)
        pltpu.sync_copy(mpy_hbm.at[pl.ds(nbase, MCH)], my_v)

        @pl.loop(0, MCH, step=L)
        def _(o):
            rows = o + iota
            plsc.store_scatter(mt_v, [rows, cols[0]], mx_v[pl.ds(o, L)])
            plsc.store_scatter(mt_v, [rows, cols[1]], my_v[pl.ds(o, L)])

        pltpu.sync_copy(mt_v, mtab_hbm.at[pl.ds(nbase, MCH)])

    for src_hbm, dst_hbm, fill in ((idxn_hbm, idxnp_hbm, TRASH),
                                   (idxc_hbm, idxcp_hbm, 0)):
        fill_vec = jnp.full((L,), fill, _i32)
        for s in range(4):
            base = wid * EPW + s * ICH
            if s == 0:
                pltpu.sync_copy(src_hbm.at[pl.ds(base, ICH)], ix_v)
            else:
                @pl.when(wid < NW - 1)
                def _():
                    pltpu.sync_copy(src_hbm.at[pl.ds(base, ICH)], ix_v)

                @pl.when(wid == NW - 1)
                def _():
                    @pl.loop(0, ICH, step=L)
                    def _(o):
                        ix_v[pl.ds(o, L)] = fill_vec

            pltpu.sync_copy(ix_v, dst_hbm.at[pl.ds(base, ICH)])


def _scatter_body(mtab_hbm, ctab_hbm, idxn_hbm, idxc_hbm, part_hbm,
                  idxn_v, idxc_v, mrows_v, crows_v, out_v, zbuf_v,
                  acc_sh, sem0, sem1):
    core = lax.axis_index("c")
    sid = lax.axis_index("s")
    wid = core * NS + sid
    iota = lax.iota(_i32, L)
    cols = [jnp.full((L,), j, _i32) for j in range(8)]

    zero = jnp.zeros((L,), _f32)
    zr = iota >> 3
    zc = iota & 7

    @pl.loop(0, ZCH * 8, step=L)
    def _(i):
        plsc.store_scatter(zbuf_v, [zr + (i >> 3), zc], zero)

    for z in range(8):
        pltpu.sync_copy(zbuf_v, acc_sh.at[pl.ds(sid * ACC_PW + z * ZCH, ZCH)])

    plsc.subcore_barrier()

    for b in range(BLOCKS):
        base = wid * EPW + b * B
        cp0 = pltpu.async_copy(idxn_hbm.at[pl.ds(base, B)], idxn_v, sem0)
        cp1 = pltpu.async_copy(idxc_hbm.at[pl.ds(base, B)], idxc_v, sem1)
        cp0.wait()
        cp1.wait()
        cp2 = pltpu.async_copy(mtab_hbm.at[idxn_v], mrows_v, sem0)
        cp3 = pltpu.async_copy(ctab_hbm.at[idxc_v], crows_v, sem1)
        cp2.wait()
        cp3.wait()

        @pl.loop(0, B, step=L)
        def _(o):
            rows = o + iota
            mpx = plsc.load_gather(mrows_v, [rows, cols[0]])
            mpy = plsc.load_gather(mrows_v, [rows, cols[1]])
            cx = plsc.load_gather(crows_v, [rows, cols[0]])
            cy = plsc.load_gather(crows_v, [rows, cols[1]])
            dx = mpx - cx
            dy = mpy - cy
            w = _rsqrt_nr(dx * dx + dy * dy)
            plsc.store_scatter(out_v, [rows, cols[3]], w)
            plsc.store_scatter(out_v, [rows, cols[4]], w * cx)
            plsc.store_scatter(out_v, [rows, cols[5]], w * cy)
            for ch in range(C):
                phi = plsc.load_gather(crows_v, [rows, cols[2 + ch]])
                plsc.store_scatter(out_v, [rows, cols[ch]], w * phi)

        pltpu.sync_copy(out_v, acc_sh.at[idxn_v], add=True)

    plsc.subcore_barrier()

    r0 = sid * ACC_PW
    pltpu.sync_copy(acc_sh.at[pl.ds(r0, ACC_PW)],
                    part_hbm.at[core, pl.ds(r0, ACC_PW)])


def _combine_body(part_hbm, mtab_hbm, g00_hbm, g01_hbm, g10_hbm, g11_hbm,
                  g20_hbm, g21_hbm, o0_hbm, o1_hbm, o2_hbm,
                  a0_v, a1_v, mt_v, gb0_v, gb1_v, gb2_v, gb3_v, gb4_v, gb5_v,
                  ob0_v, ob1_v, ob2_v, sem):
    core = lax.axis_index("c")
    sid = lax.axis_index("s")
    wid = core * NS + sid
    r0 = wid * MPW
    iota = lax.iota(_i32, L)
    cols = [jnp.full((L,), j, _i32) for j in range(8)]

    gbufs = (gb0_v, gb1_v, gb2_v, gb3_v, gb4_v, gb5_v)
    ghbms = (g00_hbm, g01_hbm, g10_hbm, g11_hbm, g20_hbm, g21_hbm)
    pend = [pltpu.async_copy(part_hbm.at[0, pl.ds(r0, MPW)], a0_v, sem),
            pltpu.async_copy(part_hbm.at[1, pl.ds(r0, MPW)], a1_v, sem),
            pltpu.async_copy(mtab_hbm.at[pl.ds(r0, MPW)], mt_v, sem)]
    for gh, gv in zip(ghbms, gbufs):
        pend.append(pltpu.async_copy(gh.at[pl.ds(r0, MPW)], gv, sem))
    for p in pend:
        p.wait()

    obufs = (ob0_v, ob1_v, ob2_v)

    @pl.loop(0, MPW, step=L)
    def _(o):
        rows = o + iota
        sw = (plsc.load_gather(a0_v, [rows, cols[3]])
              + plsc.load_gather(a1_v, [rows, cols[3]]))
        swcx = (plsc.load_gather(a0_v, [rows, cols[4]])
                + plsc.load_gather(a1_v, [rows, cols[4]]))
        swcy = (plsc.load_gather(a0_v, [rows, cols[5]])
                + plsc.load_gather(a1_v, [rows, cols[5]]))
        mpx = plsc.load_gather(mt_v, [rows, cols[0]])
        mpy = plsc.load_gather(mt_v, [rows, cols[1]])
        rsw = 1.0 / sw
        swdx = mpx * sw - swcx
        swdy = mpy * sw - swcy
        for ch in range(C):
            sphi = (plsc.load_gather(a0_v, [rows, cols[ch]])
                    + plsc.load_gather(a1_v, [rows, cols[ch]]))
            g0 = gbufs[2 * ch][pl.ds(o, L)]
            g1 = gbufs[2 * ch + 1][pl.ds(o, L)]
            num = sphi + g0 * swdx + g1 * swdy
            obufs[ch][pl.ds(o, L)] = num * rsw

    for ov, oh in zip(obufs, (o0_hbm, o1_hbm, o2_hbm)):
        pltpu.sync_copy(ov, oh.at[pl.ds(r0, MPW)])


_pack_phase = functools.partial(
    pl.kernel,
    out_type=(jax.ShapeDtypeStruct((NM, 2), _f32),
              jax.ShapeDtypeStruct((NCT, 8), _f32),
              jax.ShapeDtypeStruct((E_PAD,), _i32),
              jax.ShapeDtypeStruct((E_PAD,), _i32)),
    mesh=_mesh,
    compiler_params=_cp,
    scratch_types=[
        pltpu.VMEM((CCH,), _f32), pltpu.VMEM((CCH,), _f32),
        pltpu.VMEM((CCH,), _f32), pltpu.VMEM((CCH,), _f32),
        pltpu.VMEM((CCH,), _f32),
        pltpu.VMEM((CCH, 8), _f32),
        pltpu.VMEM((MCH,), _f32), pltpu.VMEM((MCH,), _f32),
        pltpu.VMEM((MCH, 2), _f32),
        pltpu.VMEM((ICH,), _i32),
    ],
)

_scatter_phase = functools.partial(
    pl.kernel,
    out_type=jax.ShapeDtypeStruct((NC, N_ACC, 8), _f32),
    mesh=_mesh,
    compiler_params=_cp,
    scratch_types=[
        pltpu.VMEM((B,), _i32),
        pltpu.VMEM((B,), _i32),
        pltpu.VMEM((B, 2), _f32),
        pltpu.VMEM((B, 8), _f32),
        pltpu.VMEM((B, 8), _f32),
        pltpu.VMEM((ZCH, 8), _f32),
        pltpu.VMEM_SHARED((N_ACC, 8), _f32),
        pltpu.SemaphoreType.DMA,
        pltpu.SemaphoreType.DMA,
    ],
)

_combine_phase = functools.partial(
    pl.kernel,
    out_type=(jax.ShapeDtypeStruct((NM,), _f32),
              jax.ShapeDtypeStruct((NM,), _f32),
              jax.ShapeDtypeStruct((NM,), _f32)),
    mesh=_mesh,
    compiler_params=_cp,
    scratch_types=[
        pltpu.VMEM((MPW, 8), _f32),
        pltpu.VMEM((MPW, 8), _f32),
        pltpu.VMEM((MPW, 2), _f32),
        pltpu.VMEM((MPW,), _f32), pltpu.VMEM((MPW,), _f32),
        pltpu.VMEM((MPW,), _f32), pltpu.VMEM((MPW,), _f32),
        pltpu.VMEM((MPW,), _f32), pltpu.VMEM((MPW,), _f32),
        pltpu.VMEM((MPW,), _f32),
        pltpu.VMEM((MPW,), _f32),
        pltpu.VMEM((MPW,), _f32),
        pltpu.SemaphoreType.DMA,
    ],
)


def kernel(cell_phi, cell_grad, cells_node, cells_index, centroid, mesh_pos):
    n = mesh_pos.shape[0]
    nc = centroid.shape[0]
    mpx = jnp.pad(mesh_pos[:, 0], (0, NM - n))
    mpy = jnp.pad(mesh_pos[:, 1], (0, NM - n))
    cx = jnp.pad(centroid[:, 0], (0, NCT - nc))
    cy = jnp.pad(centroid[:, 1], (0, NCT - nc))
    p0 = jnp.pad(cell_phi[:, 0], (0, NCT - nc))
    p1 = jnp.pad(cell_phi[:, 1], (0, NCT - nc))
    p2 = jnp.pad(cell_phi[:, 2], (0, NCT - nc))
    g = [cell_grad[:, ch, d] for ch in range(C) for d in range(2)]

    mtab, ctab, idxnp, idxcp = _pack_phase(_pack_body)(
        mpx, mpy, cx, cy, p0, p1, p2, cells_node, cells_index)
    acc = _scatter_phase(_scatter_body)(mtab, ctab, idxnp, idxcp)
    o0, o1, o2 = _combine_phase(_combine_body)(acc, mtab, *g)
    return jnp.stack([o0[:n], o1[:n], o2[:n]], axis=1)


# speedup vs baseline: 136.2475x; 1.2379x over previous
import dataclasses
import functools

import jax
import jax.numpy as jnp
from jax import lax
from jax.experimental import pallas as pl
from jax.experimental.pallas import tpu as pltpu
from jax.experimental.pallas import tpu_sc as plsc

NC = 2
NS = 16
NW = NC * NS
L = 16

N_NODES = 100000
N_CELLS = 200000
E = 600000
C = 3
TRASH = N_NODES

B = 1280
BLOCKS = 15
EPW = B * BLOCKS
E_PAD = EPW * NW

NM = 100352
MPW = NM // NW
N_ACC = NM
ACC_PW = N_ACC // NS
ZCH = ACC_PW // 8

NCT = 200192
CPW = NCT // NW
CCH = CPW // 2
MCH = MPW // 2
ICH = EPW // 4

_mesh = plsc.VectorSubcoreMesh(core_axis_name="c", subcore_axis_name="s")
_cp = pltpu.CompilerParams()
for _f, _v in (("needs_layout_passes", False), ("use_tc_tiling_on_sc", False)):
    if _f in pltpu.CompilerParams.__dataclass_fields__:
        _cp = dataclasses.replace(_cp, **{_f: _v})

_f32 = jnp.float32
_i32 = jnp.int32


def _rsqrt_nr(x):
    xh = x * 0.5
    i = plsc.bitcast(x, _i32)
    i = jnp.int32(0x5F3759DF) - (i >> 1)
    y = plsc.bitcast(i, _f32)
    y = y * (1.5 - xh * y * y)
    y = y * (1.5 - xh * y * y)
    y = y * (1.5 - xh * y * y)
    return y


def _pack_body(mpx_hbm, mpy_hbm, cx_hbm, cy_hbm, p0_hbm, p1_hbm, p2_hbm,
               idxn_hbm, idxc_hbm,
               mtab_hbm, ctab_hbm, idxnp_hbm, idxcp_hbm,
               c0_v, c1_v, c2_v, c3_v, c4_v, ct_v, mx_v, my_v, mt_v, ix_v,
               sema):
    core = lax.axis_index("c")
    sid = lax.axis_index("s")
    wid = core * NS + sid
    iota = lax.iota(_i32, L)
    cols = [jnp.full((L,), j, _i32) for j in range(8)]

    for s in range(2):
        cbase = wid * CPW + s * CCH
        pend = [pltpu.async_copy(src.at[pl.ds(cbase, CCH)], dst, sema)
                for src, dst in ((cx_hbm, c0_v), (cy_hbm, c1_v),
                                 (p0_hbm, c2_v), (p1_hbm, c3_v),
                                 (p2_hbm, c4_v))]
        for p in pend:
            p.wait()

        @pl.loop(0, CCH, step=L)
        def _(o):
            rows = o + iota
            for j, src_v in enumerate((c0_v, c1_v, c2_v, c3_v, c4_v)):
                plsc.store_scatter(ct_v, [rows, cols[j]], src_v[pl.ds(o, L)])

        pltpu.sync_copy(ct_v, ctab_hbm.at[pl.ds(cbase, CCH)])

    for s in range(2):
        nbase = wid * MPW + s * MCH
        pend = [pltpu.async_copy(mpx_hbm.at[pl.ds(nbase, MCH)], mx_v, sema),
                pltpu.async_copy(mpy_hbm.at[pl.ds(nbase, MCH)], my_v, sema)]
        for p in pend:
            p.wait()

        @pl.loop(0, MCH, step=L)
        def _(o):
            rows = o + iota
            plsc.store_scatter(mt_v, [rows, cols[0]], mx_v[pl.ds(o, L)])
            plsc.store_scatter(mt_v, [rows, cols[1]], my_v[pl.ds(o, L)])

        pltpu.sync_copy(mt_v, mtab_hbm.at[pl.ds(nbase, MCH)])

    for src_hbm, dst_hbm, spread in ((idxn_hbm, idxnp_hbm, True),
                                     (idxc_hbm, idxcp_hbm, False)):
        for s in range(4):
            base = wid * EPW + s * ICH
            if s == 0:
                pltpu.sync_copy(src_hbm.at[pl.ds(base, ICH)], ix_v)
            else:
                @pl.when(wid < NW - 1)
                def _():
                    pltpu.sync_copy(src_hbm.at[pl.ds(base, ICH)], ix_v)

                @pl.when(wid == NW - 1)
                def _():
                    @pl.loop(0, ICH, step=L)
                    def _(o):
                        if spread:
                            ix_v[pl.ds(o, L)] = TRASH + ((o + iota) & 255)
                        else:
                            ix_v[pl.ds(o, L)] = jnp.zeros((L,), _i32)

            pltpu.sync_copy(ix_v, dst_hbm.at[pl.ds(base, ICH)])


def _scatter_body(mtab_hbm, ctab_hbm, idxn_hbm, idxc_hbm, part_hbm,
                  idxn0_v, idxn1_v, idxn2_v, idxc0_v, idxc1_v,
                  mrows0_v, mrows1_v, crows0_v, crows1_v, out0_v, out1_v,
                  zbuf_v, acc_sh, semg0, semg1, sems0, sems1):
    core = lax.axis_index("c")
    sid = lax.axis_index("s")
    wid = core * NS + sid
    iota = lax.iota(_i32, L)
    cols = [jnp.full((L,), j, _i32) for j in range(8)]

    idxn = (idxn0_v, idxn1_v, idxn2_v)
    idxc = (idxc0_v, idxc1_v)
    mrows = (mrows0_v, mrows1_v)
    crows = (crows0_v, crows1_v)
    outs = (out0_v, out1_v)
    semg = (semg0, semg1)
    sems = (sems0, sems1)

    zero = jnp.zeros((L,), _f32)
    zr = iota >> 3
    zc = iota & 7

    @pl.loop(0, ZCH * 8, step=L)
    def _(i):
        plsc.store_scatter(zbuf_v, [zr + (i >> 3), zc], zero)

    for z in range(8):
        pltpu.sync_copy(zbuf_v, acc_sh.at[pl.ds(sid * ACC_PW + z * ZCH, ZCH)])

    plsc.subcore_barrier()

    def load_idx(b):
        base = wid * EPW + b * B
        k = b % 2
        pltpu.async_copy(idxn_hbm.at[pl.ds(base, B)], idxn[b % 3],
                         semg[k]).wait()
        pltpu.async_copy(idxc_hbm.at[pl.ds(base, B)], idxc[k], semg[k]).wait()

    def start_gather(b):
        k = b % 2
        g0 = pltpu.async_copy(mtab_hbm.at[idxn[b % 3]], mrows[k], semg[k])
        g1 = pltpu.async_copy(ctab_hbm.at[idxc[k]], crows[k], semg[k])
        return (g0, g1)

    def compute(k):
        mr, cr, ov = mrows[k], crows[k], outs[k]

        @pl.loop(0, B, step=L)
        def _(o):
            rows = o + iota
            mpx = plsc.load_gather(mr, [rows, cols[0]])
            mpy = plsc.load_gather(mr, [rows, cols[1]])
            cx = plsc.load_gather(cr, [rows, cols[0]])
            cy = plsc.load_gather(cr, [rows, cols[1]])
            dx = mpx - cx
            dy = mpy - cy
            w = _rsqrt_nr(dx * dx + dy * dy)
            plsc.store_scatter(ov, [rows, cols[3]], w)
            plsc.store_scatter(ov, [rows, cols[4]], w * cx)
            plsc.store_scatter(ov, [rows, cols[5]], w * cy)
            for ch in range(C):
                phi = plsc.load_gather(cr, [rows, cols[2 + ch]])
                plsc.store_scatter(ov, [rows, cols[ch]], w * phi)

    load_idx(0)
    pending_g = start_gather(0)
    pending_s = [None, None]
    for b in range(BLOCKS):
        k = b % 2
        if pending_s[k] is not None:
            pending_s[k].wait()
        if b + 1 < BLOCKS:
            load_idx(b + 1)
            next_g = start_gather(b + 1)
        else:
            next_g = None
        for g in pending_g:
            g.wait()
        compute(k)
        pending_s[k] = pltpu.async_copy(outs[k], acc_sh.at[idxn[b % 3]],
                                        sems[k], add=True)
        pending_g = next_g
    for p in pending_s:
        if p is not None:
            p.wait()

    plsc.subcore_barrier()

    r0 = sid * ACC_PW
    pltpu.sync_copy(acc_sh.at[pl.ds(r0, ACC_PW)],
                    part_hbm.at[core, pl.ds(r0, ACC_PW)])


def _combine_body(part_hbm, mtab_hbm, g00_hbm, g01_hbm, g10_hbm, g11_hbm,
                  g20_hbm, g21_hbm, o0_hbm, o1_hbm, o2_hbm,
                  a0_v, a1_v, mt_v, gb0_v, gb1_v, gb2_v, gb3_v, gb4_v, gb5_v,
                  ob0_v, ob1_v, ob2_v, sem):
    core = lax.axis_index("c")
    sid = lax.axis_index("s")
    wid = core * NS + sid
    r0 = wid * MPW
    iota = lax.iota(_i32, L)
    cols = [jnp.full((L,), j, _i32) for j in range(8)]

    gbufs = (gb0_v, gb1_v, gb2_v, gb3_v, gb4_v, gb5_v)
    ghbms = (g00_hbm, g01_hbm, g10_hbm, g11_hbm, g20_hbm, g21_hbm)
    pend = [pltpu.async_copy(part_hbm.at[0, pl.ds(r0, MPW)], a0_v, sem),
            pltpu.async_copy(part_hbm.at[1, pl.ds(r0, MPW)], a1_v, sem),
            pltpu.async_copy(mtab_hbm.at[pl.ds(r0, MPW)], mt_v, sem)]
    for gh, gv in zip(ghbms, gbufs):
        pend.append(pltpu.async_copy(gh.at[pl.ds(r0, MPW)], gv, sem))
    for p in pend:
        p.wait()

    obufs = (ob0_v, ob1_v, ob2_v)

    @pl.loop(0, MPW, step=L)
    def _(o):
        rows = o + iota
        sw = (plsc.load_gather(a0_v, [rows, cols[3]])
              + plsc.load_gather(a1_v, [rows, cols[3]]))
        swcx = (plsc.load_gather(a0_v, [rows, cols[4]])
                + plsc.load_gather(a1_v, [rows, cols[4]]))
        swcy = (plsc.load_gather(a0_v, [rows, cols[5]])
                + plsc.load_gather(a1_v, [rows, cols[5]]))
        mpx = plsc.load_gather(mt_v, [rows, cols[0]])
        mpy = plsc.load_gather(mt_v, [rows, cols[1]])
        rsw = 1.0 / sw
        swdx = mpx * sw - swcx
        swdy = mpy * sw - swcy
        for ch in range(C):
            sphi = (plsc.load_gather(a0_v, [rows, cols[ch]])
                    + plsc.load_gather(a1_v, [rows, cols[ch]]))
            g0 = gbufs[2 * ch][pl.ds(o, L)]
            g1 = gbufs[2 * ch + 1][pl.ds(o, L)]
            num = sphi + g0 * swdx + g1 * swdy
            obufs[ch][pl.ds(o, L)] = num * rsw

    for ov, oh in zip(obufs, (o0_hbm, o1_hbm, o2_hbm)):
        pltpu.sync_copy(ov, oh.at[pl.ds(r0, MPW)])


_pack_phase = functools.partial(
    pl.kernel,
    out_type=(jax.ShapeDtypeStruct((NM, 2), _f32),
              jax.ShapeDtypeStruct((NCT, 8), _f32),
              jax.ShapeDtypeStruct((E_PAD,), _i32),
              jax.ShapeDtypeStruct((E_PAD,), _i32)),
    mesh=_mesh,
    compiler_params=_cp,
    scratch_types=[
        pltpu.VMEM((CCH,), _f32), pltpu.VMEM((CCH,), _f32),
        pltpu.VMEM((CCH,), _f32), pltpu.VMEM((CCH,), _f32),
        pltpu.VMEM((CCH,), _f32),
        pltpu.VMEM((CCH, 8), _f32),
        pltpu.VMEM((MCH,), _f32), pltpu.VMEM((MCH,), _f32),
        pltpu.VMEM((MCH, 2), _f32),
        pltpu.VMEM((ICH,), _i32),
        pltpu.SemaphoreType.DMA,
    ],
)

_scatter_phase = functools.partial(
    pl.kernel,
    out_type=jax.ShapeDtypeStruct((NC, N_ACC, 8), _f32),
    mesh=_mesh,
    compiler_params=_cp,
    scratch_types=[
        pltpu.VMEM((B,), _i32),
        pltpu.VMEM((B,), _i32),
        pltpu.VMEM((B,), _i32),
        pltpu.VMEM((B,), _i32),
        pltpu.VMEM((B,), _i32),
        pltpu.VMEM((B, 2), _f32),
        pltpu.VMEM((B, 2), _f32),
        pltpu.VMEM((B, 8), _f32),
        pltpu.VMEM((B, 8), _f32),
        pltpu.VMEM((B, 8), _f32),
        pltpu.VMEM((B, 8), _f32),
        pltpu.VMEM((ZCH, 8), _f32),
        pltpu.VMEM_SHARED((N_ACC, 8), _f32),
        pltpu.SemaphoreType.DMA,
        pltpu.SemaphoreType.DMA,
        pltpu.SemaphoreType.DMA,
        pltpu.SemaphoreType.DMA,
    ],
)

_combine_phase = functools.partial(
    pl.kernel,
    out_type=(jax.ShapeDtypeStruct((NM,), _f32),
              jax.ShapeDtypeStruct((NM,), _f32),
              jax.ShapeDtypeStruct((NM,), _f32)),
    mesh=_mesh,
    compiler_params=_cp,
    scratch_types=[
        pltpu.VMEM((MPW, 8), _f32),
        pltpu.VMEM((MPW, 8), _f32),
        pltpu.VMEM((MPW, 2), _f32),
        pltpu.VMEM((MPW,), _f32), pltpu.VMEM((MPW,), _f32),
        pltpu.VMEM((MPW,), _f32), pltpu.VMEM((MPW,), _f32),
        pltpu.VMEM((MPW,), _f32), pltpu.VMEM((MPW,), _f32),
        pltpu.VMEM((MPW,), _f32),
        pltpu.VMEM((MPW,), _f32),
        pltpu.VMEM((MPW,), _f32),
        pltpu.SemaphoreType.DMA,
    ],
)


def kernel(cell_phi, cell_grad, cells_node, cells_index, centroid, mesh_pos):
    n = mesh_pos.shape[0]
    nc = centroid.shape[0]
    mpx = jnp.pad(mesh_pos[:, 0], (0, NM - n))
    mpy = jnp.pad(mesh_pos[:, 1], (0, NM - n))
    cx = jnp.pad(centroid[:, 0], (0, NCT - nc))
    cy = jnp.pad(centroid[:, 1], (0, NCT - nc))
    p0 = jnp.pad(cell_phi[:, 0], (0, NCT - nc))
    p1 = jnp.pad(cell_phi[:, 1], (0, NCT - nc))
    p2 = jnp.pad(cell_phi[:, 2], (0, NCT - nc))
    g = [cell_grad[:, ch, d] for ch in range(C) for d in range(2)]

    mtab, ctab, idxnp, idxcp = _pack_phase(_pack_body)(
        mpx, mpy, cx, cy, p0, p1, p2, cells_node, cells_index)
    acc = _scatter_phase(_scatter_body)(mtab, ctab, idxnp, idxcp)
    o0, o1, o2 = _combine_phase(_combine_body)(acc, mtab, *g)
    return jnp.stack([o0[:n], o1[:n], o2[:n]], axis=1)


# speedup vs baseline: 139.4228x; 1.0233x over previous
import dataclasses
import functools

import jax
import jax.numpy as jnp
from jax import lax
from jax.experimental import pallas as pl
from jax.experimental.pallas import tpu as pltpu
from jax.experimental.pallas import tpu_sc as plsc

NC = 2
NS = 16
NW = NC * NS
L = 16

N_NODES = 100000
N_CELLS = 200000
E = 600000
C = 3
TRASH = N_NODES

B = 1280
BLOCKS = 15
EPW = B * BLOCKS
E_PAD = EPW * NW

NM = 100352
MPW = NM // NW
N_ACC = NM
ACC_PW = N_ACC // NS
ZCH = ACC_PW // 16

NCT = 200192
CPW = NCT // NW
CCH = CPW // 2
MCH = MPW // 2
ICH = EPW // 4
RPW = 18752
RPW_LAST = E - (NW - 1) * RPW
TAIL = RPW - 3 * ICH
TAIL_LAST = RPW_LAST - 3 * ICH

_mesh = plsc.VectorSubcoreMesh(core_axis_name="c", subcore_axis_name="s")
_cp = pltpu.CompilerParams()
for _f, _v in (("needs_layout_passes", False), ("use_tc_tiling_on_sc", False)):
    if _f in pltpu.CompilerParams.__dataclass_fields__:
        _cp = dataclasses.replace(_cp, **{_f: _v})

_f32 = jnp.float32
_i32 = jnp.int32


def _rsqrt_nr(x):
    xh = x * 0.5
    i = plsc.bitcast(x, _i32)
    i = jnp.int32(0x5F3759DF) - (i >> 1)
    y = plsc.bitcast(i, _f32)
    y = y * (1.5 - xh * y * y)
    y = y * (1.5 - xh * y * y)
    y = y * (1.5 - xh * y * y)
    return y


def _pack_body(mpx_hbm, mpy_hbm, cx_hbm, cy_hbm, p0_hbm, p1_hbm, p2_hbm,
               idxn_hbm, idxc_hbm,
               mtab_hbm, ctab_hbm, idxnp_hbm, idxcp_hbm,
               c0_v, c1_v, c2_v, c3_v, c4_v, ct_v, mx_v, my_v, mt_v, ix_v,
               sema):
    core = lax.axis_index("c")
    sid = lax.axis_index("s")
    wid = core * NS + sid
    iota = lax.iota(_i32, L)
    cols = [jnp.full((L,), j, _i32) for j in range(8)]

    for s in range(2):
        cbase = wid * CPW + s * CCH
        pend = [pltpu.async_copy(src.at[pl.ds(cbase, CCH)], dst, sema)
                for src, dst in ((cx_hbm, c0_v), (cy_hbm, c1_v),
                                 (p0_hbm, c2_v), (p1_hbm, c3_v),
                                 (p2_hbm, c4_v))]
        for p in pend:
            p.wait()

        @pl.loop(0, CCH, step=L)
        def _(o):
            rows = o + iota
            for j, src_v in enumerate((c0_v, c1_v, c2_v, c3_v, c4_v)):
                plsc.store_scatter(ct_v, [rows, cols[j]], src_v[pl.ds(o, L)])

        pltpu.sync_copy(ct_v, ctab_hbm.at[pl.ds(cbase, CCH)])

    for s in range(2):
        nbase = wid * MPW + s * MCH
        pend = [pltpu.async_copy(mpx_hbm.at[pl.ds(nbase, MCH)], mx_v, sema),
                pltpu.async_copy(mpy_hbm.at[pl.ds(nbase, MCH)], my_v, sema)]
        for p in pend:
            p.wait()

        @pl.loop(0, MCH, step=L)
        def _(o):
            rows = o + iota
            plsc.store_scatter(mt_v, [rows, cols[0]], mx_v[pl.ds(o, L)])
            plsc.store_scatter(mt_v, [rows, cols[1]], my_v[pl.ds(o, L)])

        pltpu.sync_copy(mt_v, mtab_hbm.at[pl.ds(nbase, MCH)])

    for src_hbm, dst_hbm, spread in ((idxn_hbm, idxnp_hbm, True),
                                     (idxc_hbm, idxcp_hbm, False)):
        def fill(lo):
            @pl.loop(lo, ICH, step=L)
            def _(o):
                if spread:
                    ix_v[pl.ds(o, L)] = TRASH + ((o + iota) & 255)
                else:
                    ix_v[pl.ds(o, L)] = jnp.zeros((L,), _i32)

        for s in range(4):
            dbase = wid * EPW + s * ICH
            sbase = wid * RPW + s * ICH
            if s < 3:
                pltpu.sync_copy(src_hbm.at[pl.ds(sbase, ICH)], ix_v)
            else:
                @pl.when(wid < NW - 1)
                def _():
                    pltpu.sync_copy(src_hbm.at[pl.ds(sbase, TAIL)],
                                    ix_v.at[pl.ds(0, TAIL)])
                    fill(TAIL)

                @pl.when(wid == NW - 1)
                def _():
                    lbase = (NW - 1) * RPW + s * ICH
                    pltpu.sync_copy(src_hbm.at[pl.ds(lbase, TAIL_LAST)],
                                    ix_v.at[pl.ds(0, TAIL_LAST)])
                    fill(TAIL_LAST)

            pltpu.sync_copy(ix_v, dst_hbm.at[pl.ds(dbase, ICH)])


def _scatter_body(mtab_hbm, ctab_hbm, idxn_hbm, idxc_hbm, part_hbm,
                  idxn0_v, idxn1_v, idxn2_v, idxn3_v,
                  idxc0_v, idxc1_v, idxc2_v, idxc3_v,
                  mrows0_v, mrows1_v, crows0_v, crows1_v, out0_v, out1_v,
                  zbuf_v, acc_sh, semi, semg0, semg1, sems0, sems1):
    core = lax.axis_index("c")
    sid = lax.axis_index("s")
    wid = core * NS + sid
    iota = lax.iota(_i32, L)
    cols = [jnp.full((L,), j, _i32) for j in range(8)]

    idxn = (idxn0_v, idxn1_v, idxn2_v, idxn3_v)
    idxc = (idxc0_v, idxc1_v, idxc2_v, idxc3_v)
    mrows = (mrows0_v, mrows1_v)
    crows = (crows0_v, crows1_v)
    outs = (out0_v, out1_v)
    semg = (semg0, semg1)
    sems = (sems0, sems1)

    zero = jnp.zeros((L,), _f32)
    zr = iota >> 3
    zc = iota & 7

    @pl.loop(0, ZCH * 8, step=L)
    def _(i):
        plsc.store_scatter(zbuf_v, [zr + (i >> 3), zc], zero)

    for z in range(16):
        pltpu.sync_copy(zbuf_v, acc_sh.at[pl.ds(sid * ACC_PW + z * ZCH, ZCH)])

    plsc.subcore_barrier()

    def load_idx(b):
        base = wid * EPW + b * B
        return (pltpu.async_copy(idxn_hbm.at[pl.ds(base, B)], idxn[b % 4],
                                 semi),
                pltpu.async_copy(idxc_hbm.at[pl.ds(base, B)], idxc[b % 4],
                                 semi))

    def start_gather(b):
        k = b % 2
        g0 = pltpu.async_copy(mtab_hbm.at[idxn[b % 4]], mrows[k], semg[k])
        g1 = pltpu.async_copy(ctab_hbm.at[idxc[b % 4]], crows[k], semg[k])
        return (g0, g1)

    def compute(k):
        mr, cr, ov = mrows[k], crows[k], outs[k]

        @plsc.parallel_loop(0, B, L, unroll=2)
        def _(o):
            rows = o + iota
            mpx = plsc.load_gather(mr, [rows, cols[0]])
            mpy = plsc.load_gather(mr, [rows, cols[1]])
            cx = plsc.load_gather(cr, [rows, cols[0]])
            cy = plsc.load_gather(cr, [rows, cols[1]])
            dx = mpx - cx
            dy = mpy - cy
            w = _rsqrt_nr(dx * dx + dy * dy)
            plsc.store_scatter(ov, [rows, cols[3]], w)
            plsc.store_scatter(ov, [rows, cols[4]], w * cx)
            plsc.store_scatter(ov, [rows, cols[5]], w * cy)
            for ch in range(C):
                phi = plsc.load_gather(cr, [rows, cols[2 + ch]])
                plsc.store_scatter(ov, [rows, cols[ch]], w * phi)

    pend_i = {0: load_idx(0)}
    if BLOCKS > 1:
        pend_i[1] = load_idx(1)
    for p in pend_i[0]:
        p.wait()
    pending_g = start_gather(0)
    pending_s = [None, None]
    for b in range(BLOCKS):
        k = b % 2
        if pending_s[k] is not None:
            pending_s[k].wait()
        if b + 2 < BLOCKS:
            pend_i[b + 2] = load_idx(b + 2)
        if b + 1 < BLOCKS:
            for p in pend_i.pop(b + 1):
                p.wait()
            next_g = start_gather(b + 1)
        else:
            next_g = None
        for g in pending_g:
            g.wait()
        compute(k)
        pending_s[k] = pltpu.async_copy(outs[k], acc_sh.at[idxn[b % 4]],
                                        sems[k], add=True)
        pending_g = next_g
    for p in pending_s:
        if p is not None:
            p.wait()

    plsc.subcore_barrier()

    r0 = sid * ACC_PW
    pltpu.sync_copy(acc_sh.at[pl.ds(r0, ACC_PW)],
                    part_hbm.at[core, pl.ds(r0, ACC_PW)])


def _combine_body(part_hbm, mtab_hbm, g00_hbm, g01_hbm, g10_hbm, g11_hbm,
                  g20_hbm, g21_hbm, o0_hbm, o1_hbm, o2_hbm,
                  a0_v, a1_v, mt_v, gb0_v, gb1_v, gb2_v, gb3_v, gb4_v, gb5_v,
                  ob0_v, ob1_v, ob2_v, sem):
    core = lax.axis_index("c")
    sid = lax.axis_index("s")
    wid = core * NS + sid
    r0 = wid * MPW
    iota = lax.iota(_i32, L)
    cols = [jnp.full((L,), j, _i32) for j in range(8)]

    gbufs = (gb0_v, gb1_v, gb2_v, gb3_v, gb4_v, gb5_v)
    ghbms = (g00_hbm, g01_hbm, g10_hbm, g11_hbm, g20_hbm, g21_hbm)
    pend = [pltpu.async_copy(part_hbm.at[0, pl.ds(r0, MPW)], a0_v, sem),
            pltpu.async_copy(part_hbm.at[1, pl.ds(r0, MPW)], a1_v, sem),
            pltpu.async_copy(mtab_hbm.at[pl.ds(r0, MPW)], mt_v, sem)]
    for gh, gv in zip(ghbms, gbufs):
        pend.append(pltpu.async_copy(gh.at[pl.ds(r0, MPW)], gv, sem))
    for p in pend:
        p.wait()

    obufs = (ob0_v, ob1_v, ob2_v)

    @pl.loop(0, MPW, step=L)
    def _(o):
        rows = o + iota
        sw = (plsc.load_gather(a0_v, [rows, cols[3]])
              + plsc.load_gather(a1_v, [rows, cols[3]]))
        swcx = (plsc.load_gather(a0_v, [rows, cols[4]])
                + plsc.load_gather(a1_v, [rows, cols[4]]))
        swcy = (plsc.load_gather(a0_v, [rows, cols[5]])
                + plsc.load_gather(a1_v, [rows, cols[5]]))
        mpx = plsc.load_gather(mt_v, [rows, cols[0]])
        mpy = plsc.load_gather(mt_v, [rows, cols[1]])
        rsw = 1.0 / sw
        swdx = mpx * sw - swcx
        swdy = mpy * sw - swcy
        for ch in range(C):
            sphi = (plsc.load_gather(a0_v, [rows, cols[ch]])
                    + plsc.load_gather(a1_v, [rows, cols[ch]]))
            g0 = gbufs[2 * ch][pl.ds(o, L)]
            g1 = gbufs[2 * ch + 1][pl.ds(o, L)]
            num = sphi + g0 * swdx + g1 * swdy
            obufs[ch][pl.ds(o, L)] = num * rsw

    for ov, oh in zip(obufs, (o0_hbm, o1_hbm, o2_hbm)):
        pltpu.sync_copy(ov, oh.at[pl.ds(r0, MPW)])


_pack_phase = functools.partial(
    pl.kernel,
    out_type=(jax.ShapeDtypeStruct((NM, 2), _f32),
              jax.ShapeDtypeStruct((NCT, 8), _f32),
              jax.ShapeDtypeStruct((E_PAD,), _i32),
              jax.ShapeDtypeStruct((E_PAD,), _i32)),
    mesh=_mesh,
    compiler_params=_cp,
    scratch_types=[
        pltpu.VMEM((CCH,), _f32), pltpu.VMEM((CCH,), _f32),
        pltpu.VMEM((CCH,), _f32), pltpu.VMEM((CCH,), _f32),
        pltpu.VMEM((CCH,), _f32),
        pltpu.VMEM((CCH, 8), _f32),
        pltpu.VMEM((MCH,), _f32), pltpu.VMEM((MCH,), _f32),
        pltpu.VMEM((MCH, 2), _f32),
        pltpu.VMEM((ICH,), _i32),
        pltpu.SemaphoreType.DMA,
    ],
)

_scatter_phase = functools.partial(
    pl.kernel,
    out_type=jax.ShapeDtypeStruct((NC, N_ACC, 8), _f32),
    mesh=_mesh,
    compiler_params=_cp,
    scratch_types=(
        [pltpu.VMEM((B,), _i32)] * 8
        + [pltpu.VMEM((B, 2), _f32)] * 2
        + [pltpu.VMEM((B, 8), _f32)] * 4
        + [pltpu.VMEM((ZCH, 8), _f32),
           pltpu.VMEM_SHARED((N_ACC, 8), _f32)]
        + [pltpu.SemaphoreType.DMA] * 5
    ),
)

_combine_phase = functools.partial(
    pl.kernel,
    out_type=(jax.ShapeDtypeStruct((NM,), _f32),
              jax.ShapeDtypeStruct((NM,), _f32),
              jax.ShapeDtypeStruct((NM,), _f32)),
    mesh=_mesh,
    compiler_params=_cp,
    scratch_types=[
        pltpu.VMEM((MPW, 8), _f32),
        pltpu.VMEM((MPW, 8), _f32),
        pltpu.VMEM((MPW, 2), _f32),
        pltpu.VMEM((MPW,), _f32), pltpu.VMEM((MPW,), _f32),
        pltpu.VMEM((MPW,), _f32), pltpu.VMEM((MPW,), _f32),
        pltpu.VMEM((MPW,), _f32), pltpu.VMEM((MPW,), _f32),
        pltpu.VMEM((MPW,), _f32),
        pltpu.VMEM((MPW,), _f32),
        pltpu.VMEM((MPW,), _f32),
        pltpu.SemaphoreType.DMA,
    ],
)


def kernel(cell_phi, cell_grad, cells_node, cells_index, centroid, mesh_pos):
    n = mesh_pos.shape[0]
    nc = centroid.shape[0]
    mpx = jnp.pad(mesh_pos[:, 0], (0, NM - n))
    mpy = jnp.pad(mesh_pos[:, 1], (0, NM - n))
    cx = jnp.pad(centroid[:, 0], (0, NCT - nc))
    cy = jnp.pad(centroid[:, 1], (0, NCT - nc))
    p0 = jnp.pad(cell_phi[:, 0], (0, NCT - nc))
    p1 = jnp.pad(cell_phi[:, 1], (0, NCT - nc))
    p2 = jnp.pad(cell_phi[:, 2], (0, NCT - nc))
    g = [cell_grad[:, ch, d] for ch in range(C) for d in range(2)]

    mtab, ctab, idxnp, idxcp = _pack_phase(_pack_body)(
        mpx, mpy, cx, cy, p0, p1, p2, cells_node, cells_index)
    acc = _scatter_phase(_scatter_body)(mtab, ctab, idxnp, idxcp)
    o0, o1, o2 = _combine_phase(_combine_body)(acc, mtab, *g)
    return jnp.stack([o0[:n], o1[:n], o2[:n]], axis=1)
